# baseline (speedup 1.0000x reference)
import functools

import jax
import jax.numpy as jnp
from jax import lax
from jax.experimental import pallas as pl
from jax.experimental.pallas import tpu as pltpu
from jax.experimental.pallas import tpu_sc as plsc

B, S, H, D = 8, 2048, 8, 128
ROWS = B * S
NW = 32
RPW = ROWS // NW
CHUNK_OFFS = tuple(range(0, 504, 56)) + (504,)
CHUNK_SIZES = (56,) * 9 + (8,)
NCHUNK = len(CHUNK_OFFS)


def _tc_body(len_ref, ck, kt, ok):
    b = pl.program_id(0)
    ok[...] = ck[...]
    l = len_ref[b]
    ok[0, pl.ds(l, 1)] = kt[pl.ds(b, 1), 0]


def _tc_key(cached_key, key_token, lengths):
    out_sds = jax.ShapeDtypeStruct((B, S, H, D), jnp.float32)
    cache_spec = pl.BlockSpec((1, S, H, D), lambda b: (b, 0, 0, 0))
    token_spec = pl.BlockSpec((B, 1, H, D), lambda b: (0, 0, 0, 0))
    return pl.pallas_call(
        _tc_body,
        grid=(B,),
        in_specs=[
            pl.BlockSpec(memory_space=pltpu.SMEM),
            cache_spec,
            token_spec,
        ],
        out_specs=cache_spec,
        out_shape=out_sds,
        compiler_params=pltpu.CompilerParams(
            dimension_semantics=("parallel",),
            vmem_limit_bytes=60 * 1024 * 1024,
        ),
    )(lengths, cached_key, key_token)


_SC_MESH = plsc.VectorSubcoreMesh(core_axis_name="c", subcore_axis_name="s")


@functools.partial(
    pl.kernel,
    out_type=jax.ShapeDtypeStruct((ROWS, H * D), jnp.float32),
    mesh=_SC_MESH,
    compiler_params=pltpu.CompilerParams(needs_layout_passes=False),
    scratch_types=[
        pltpu.VMEM((2, 56, H * D), jnp.float32),
        pltpu.VMEM((16,), jnp.int32),
        pltpu.SemaphoreType.DMA,
        pltpu.SemaphoreType.DMA,
    ],
)
def _sc_value(cv2d, vt2d, len16, out, ring, len_v, sin, sout):
    cid = lax.axis_index("c")
    sid = lax.axis_index("s")
    wid = sid * 2 + cid
    base = wid * RPW

    pltpu.sync_copy(len16, len_v)

    def _load(i):
        n = CHUNK_SIZES[i]
        return pltpu.make_async_copy(
            cv2d.at[pl.ds(base + CHUNK_OFFS[i], n)],
            ring.at[i % 2, pl.ds(0, n)],
            sin,
        )

    def _store(i):
        n = CHUNK_SIZES[i]
        return pltpu.make_async_copy(
            ring.at[i % 2, pl.ds(0, n)],
            out.at[pl.ds(base + CHUNK_OFFS[i], n)],
            sout,
        )

    _load(0).start()
    for i in range(NCHUNK):
        if i + 1 < NCHUNK:
            if i >= 1:
                _store(i - 1).wait()
            _load(i + 1).start()
        _load(i).wait()
        _store(i).start()
    for i in range(NCHUNK - 2, NCHUNK):
        _store(i).wait()

    pltpu.sync_copy(vt2d, ring.at[0, pl.ds(0, B)])
    lens = len_v[...]
    lane = lax.broadcasted_iota(jnp.int32, (16,), 0)
    for b in range(B):
        l_b = jnp.max(jnp.where(lane == b, lens, -1))
        row = b * S + l_b

        @pl.when((row >= base) & (row < base + RPW))
        def _(row=row, b=b):
            pltpu.sync_copy(ring.at[0, pl.ds(b, 1)], out.at[pl.ds(row, 1)])


def kernel(cached_key, cached_value, key_token, value_token, lengths):
    len16 = jnp.concatenate([lengths, jnp.zeros((8,), jnp.int32)])
    new_value = _sc_value(
        cached_value.reshape(ROWS, H * D),
        value_token.reshape(B, H * D),
        len16,
    )
    new_key = _tc_key(cached_key, key_token, lengths)
    return (new_key, new_value.reshape(B, S, H, D))


# speedup vs baseline: 1.0012x; 1.0012x over previous
import functools

import jax
import jax.numpy as jnp
from jax import lax
from jax.experimental import pallas as pl
from jax.experimental.pallas import tpu as pltpu
from jax.experimental.pallas import tpu_sc as plsc

B, S, H, D = 8, 2048, 8, 128
ROWS = B * S
NW = 32
RPW = ROWS // NW
CHUNK_OFFS = tuple(range(0, 504, 56)) + (504,)
CHUNK_SIZES = (56,) * 9 + (8,)
NCHUNK = len(CHUNK_OFFS)


def _tc_body(len_ref, ck, kt, ok):
    b = pl.program_id(0)
    ok[...] = ck[...]
    l = len_ref[b]
    ok[0, pl.ds(l, 1)] = kt[pl.ds(b, 1), 0]


def _tc_key(cached_key, key_token, lengths):
    out_sds = jax.ShapeDtypeStruct((B, S, H, D), jnp.float32)
    cache_spec = pl.BlockSpec((1, S, H, D), lambda b: (b, 0, 0, 0))
    token_spec = pl.BlockSpec((B, 1, H, D), lambda b: (0, 0, 0, 0))
    return pl.pallas_call(
        _tc_body,
        grid=(B,),
        in_specs=[
            pl.BlockSpec(memory_space=pltpu.SMEM),
            cache_spec,
            token_spec,
        ],
        out_specs=cache_spec,
        out_shape=out_sds,
        compiler_params=pltpu.CompilerParams(
            dimension_semantics=("parallel",),
            vmem_limit_bytes=60 * 1024 * 1024,
        ),
    )(lengths, cached_key, key_token)


_SC_MESH = plsc.VectorSubcoreMesh(core_axis_name="c", subcore_axis_name="s")


@functools.partial(
    pl.kernel,
    out_type=jax.ShapeDtypeStruct((ROWS, H * D), jnp.float32),
    mesh=_SC_MESH,
    compiler_params=pltpu.CompilerParams(needs_layout_passes=False),
    scratch_types=[
        pltpu.VMEM_SHARED((16, 2, 56, H * D), jnp.float32),
        pltpu.VMEM((16,), jnp.int32),
        pltpu.SemaphoreType.DMA,
        pltpu.SemaphoreType.DMA,
    ],
)
def _sc_value(cv2d, vt2d, len16, out, ring, len_v, sin, sout):
    cid = lax.axis_index("c")
    sid = lax.axis_index("s")
    wid = sid * 2 + cid
    base = wid * RPW

    pltpu.sync_copy(len16, len_v)

    def _load(i):
        n = CHUNK_SIZES[i]
        return pltpu.make_async_copy(
            cv2d.at[pl.ds(base + CHUNK_OFFS[i], n)],
            ring.at[sid, i % 2, pl.ds(0, n)],
            sin,
        )

    def _store(i):
        n = CHUNK_SIZES[i]
        return pltpu.make_async_copy(
            ring.at[sid, i % 2, pl.ds(0, n)],
            out.at[pl.ds(base + CHUNK_OFFS[i], n)],
            sout,
        )

    _load(0).start()
    for i in range(NCHUNK):
        if i + 1 < NCHUNK:
            if i >= 1:
                _store(i - 1).wait()
            _load(i + 1).start()
        _load(i).wait()
        _store(i).start()
    for i in range(NCHUNK - 2, NCHUNK):
        _store(i).wait()

    pltpu.sync_copy(vt2d, ring.at[sid, 0, pl.ds(0, B)])
    lens = len_v[...]
    lane = lax.broadcasted_iota(jnp.int32, (16,), 0)
    for b in range(B):
        l_b = jnp.max(jnp.where(lane == b, lens, -1))
        row = b * S + l_b

        @pl.when((row >= base) & (row < base + RPW))
        def _(row=row, b=b):
            pltpu.sync_copy(ring.at[sid, 0, pl.ds(b, 1)], out.at[pl.ds(row, 1)])


def kernel(cached_key, cached_value, key_token, value_token, lengths):
    len16 = jnp.concatenate([lengths, jnp.zeros((8,), jnp.int32)])
    new_value = _sc_value(
        cached_value.reshape(ROWS, H * D),
        value_token.reshape(B, H * D),
        len16,
    )
    new_key = _tc_key(cached_key, key_token, lengths)
    return (new_key, new_value.reshape(B, S, H, D))


# speedup vs baseline: 2.5025x; 2.4994x over previous
import jax
import jax.numpy as jnp
from jax.experimental import pallas as pl
from jax.experimental.pallas import tpu as pltpu

B, S, H, D = 8, 2048, 8, 128
S_CHUNKS = 2
CS = S // S_CHUNKS


def _kv_append_kernel(len_ref, ck, cv, kt, vt, ok, ov):
    b = pl.program_id(1)
    c = pl.program_id(0)
    ok[...] = ck[...]
    ov[...] = cv[...]
    l = len_ref[b]
    base = c * CS

    @pl.when((l >= base) & (l < base + CS))
    def _():
        r = l - base
        ok[0, pl.ds(r, 1)] = kt[pl.ds(b, 1), 0]
        ov[0, pl.ds(r, 1)] = vt[pl.ds(b, 1), 0]


def kernel(cached_key, cached_value, key_token, value_token, lengths):
    out_sds = jax.ShapeDtypeStruct((B, S, H, D), jnp.float32)
    cache_spec = pl.BlockSpec((1, CS, H, D), lambda c, b: (b, c, 0, 0))
    token_spec = pl.BlockSpec((B, 1, H, D), lambda c, b: (0, 0, 0, 0))
    new_key, new_value = pl.pallas_call(
        _kv_append_kernel,
        grid=(S_CHUNKS, B),
        in_specs=[
            pl.BlockSpec(memory_space=pltpu.SMEM),
            cache_spec,
            cache_spec,
            token_spec,
            token_spec,
        ],
        out_specs=[cache_spec, cache_spec],
        out_shape=[out_sds, out_sds],
        compiler_params=pltpu.CompilerParams(
            dimension_semantics=("parallel", "parallel"),
            vmem_limit_bytes=100 * 1024 * 1024,
        ),
    )(lengths, cached_key, cached_value, key_token, value_token)
    return (new_key, new_value)
